# baseline (speedup 1.0000x reference)
import functools

import jax
import jax.numpy as jnp
from jax import lax
from jax.experimental import pallas as pl
from jax.experimental.pallas import tpu as pltpu
from jax.experimental.pallas import tpu_sc as plsc

D = 64
B = 1024
L = 50
N_IDX = B * L

_info = plsc.get_sparse_core_info()
_NC, _NS = _info.num_cores, _info.num_subcores
_NW = _NC * _NS
_B_PER_W = N_IDX // _NW
_CHUNK = 400
_N_CHUNKS = _B_PER_W // _CHUNK


def _item_gather_sc():
    mesh = plsc.VectorSubcoreMesh(core_axis_name="c", subcore_axis_name="s")

    @functools.partial(
        pl.kernel,
        mesh=mesh,
        out_type=jax.ShapeDtypeStruct((N_IDX, D), jnp.float32),
        scratch_types=[
            pltpu.VMEM((_B_PER_W,), jnp.int32),
            pltpu.VMEM((_CHUNK, D), jnp.float32),
            pltpu.VMEM((_CHUNK, D), jnp.float32),
            pltpu.SemaphoreType.DMA,
            pltpu.SemaphoreType.DMA,
        ],
        compiler_params=pltpu.CompilerParams(use_tc_tiling_on_sc=False),
    )
    def gather_item(idx_hbm, table_hbm, out_hbm, idx_v, rows0, rows1, gsem, ssem):
        wid = lax.axis_index("s") * _NC + lax.axis_index("c")
        base = wid * _B_PER_W
        pltpu.sync_copy(idx_hbm.at[pl.ds(base, _B_PER_W)], idx_v)
        bufs = (rows0, rows1)
        gathers = [None] * _N_CHUNKS
        scatters = [None] * _N_CHUNKS
        gathers[0] = pltpu.async_copy(
            table_hbm.at[idx_v.at[pl.ds(0, _CHUNK)]], bufs[0], gsem
        )
        for c in range(_N_CHUNKS):
            buf = bufs[c % 2]
            gathers[c].wait()
            scatters[c] = pltpu.async_copy(
                buf, out_hbm.at[pl.ds(base + c * _CHUNK, _CHUNK)], ssem
            )
            nxt = c + 1
            if nxt < _N_CHUNKS:
                if nxt >= 2:
                    scatters[nxt - 2].wait()
                gathers[nxt] = pltpu.async_copy(
                    table_hbm.at[idx_v.at[pl.ds(nxt * _CHUNK, _CHUNK)]],
                    bufs[nxt % 2],
                    gsem,
                )
        scatters[_N_CHUNKS - 2].wait()
        scatters[_N_CHUNKS - 1].wait()

    return gather_item


_L_BLK = 5


def _small_lookup_tc_body(ha_i, ma_i, sa_i, hb_i, mb_i, sb_i,
                          ht_a, mt_a, st_a, ht_b, mt_b, st_b,
                          o1, o2, o3, o4, o5, o6):
    pairs = [
        (ha_i, ht_a, o1), (ma_i, mt_a, o2), (sa_i, st_a, o3),
        (hb_i, ht_b, o4), (mb_i, mt_b, o5), (sb_i, st_b, o6),
    ]
    l0 = pl.program_id(0) * _L_BLK
    for idx_ref, tab_ref, out_ref in pairs:
        t_rows = tab_ref.shape[0]
        tab = tab_ref[...]
        iota = lax.broadcasted_iota(jnp.int32, (t_rows, B), 0)
        for j in range(_L_BLK):
            row = idx_ref[l0 + j, :]
            onehot = jnp.where(iota == row[None, :], 1.0, 0.0).astype(jnp.float32)
            out_ref[j, :, :] = lax.dot_general(
                tab, onehot,
                ((( 0,), (0,)), ((), ())),
                preferred_element_type=jnp.float32,
                precision=lax.Precision.HIGHEST,
            )


def _small_lookup_tc(idxTs, tables):
    idx_spec = pl.BlockSpec((L, B), lambda i: (0, 0))
    tab_specs = [
        pl.BlockSpec((t.shape[0], D), lambda i: (0, 0)) for t in tables
    ]
    out_spec = pl.BlockSpec((_L_BLK, D, B), lambda i: (i, 0, 0))
    out_struct = jax.ShapeDtypeStruct((L, D, B), jnp.float32)
    return pl.pallas_call(
        _small_lookup_tc_body,
        grid=(L // _L_BLK,),
        in_specs=[idx_spec] * 6 + tab_specs,
        out_specs=[out_spec] * 6,
        out_shape=[out_struct] * 6,
    )(*idxTs, *tables)


def kernel(session, h_a_o, m_a_o, s_a_o, h_b_o, m_b_o, s_b_o,
           item_table, hour_table_a, minute_table_a, second_table_a,
           hour_table_b, minute_table_b, second_table_b):
    item_out = _item_gather_sc()(session.reshape(N_IDX), item_table)

    idxTs = [a.T for a in (h_a_o, m_a_o, s_a_o, h_b_o, m_b_o, s_b_o)]
    tables = [hour_table_a, minute_table_a, second_table_a,
              hour_table_b, minute_table_b, second_table_b]
    small = _small_lookup_tc(idxTs, tables)

    outs = [item_out.reshape(B, L, D)]
    outs += [jnp.transpose(y, (2, 0, 1)) for y in small]
    return tuple(outs)


# speedup vs baseline: 1.0850x; 1.0850x over previous
import functools

import jax
import jax.numpy as jnp
from jax import lax
from jax.experimental import pallas as pl
from jax.experimental.pallas import tpu as pltpu
from jax.experimental.pallas import tpu_sc as plsc

D = 64
B = 1024
L = 50
N_IDX = B * L

_info = plsc.get_sparse_core_info()
_NC, _NS = _info.num_cores, _info.num_subcores
_NW = _NC * _NS
_B_PER_W = N_IDX // _NW
_CHUNK = 400
_N_CHUNKS = _B_PER_W // _CHUNK


_DP = 128


def _item_gather_sc():
    mesh = plsc.VectorSubcoreMesh(core_axis_name="c", subcore_axis_name="s")

    @functools.partial(
        pl.kernel,
        mesh=mesh,
        out_type=jax.ShapeDtypeStruct((N_IDX, D), jnp.float32),
        scratch_types=[
            pltpu.VMEM((_B_PER_W,), jnp.int32),
            pltpu.VMEM((_CHUNK, _DP), jnp.float32),
            pltpu.VMEM((_CHUNK, _DP), jnp.float32),
            pltpu.SemaphoreType.DMA,
            pltpu.SemaphoreType.DMA,
        ],
        compiler_params=pltpu.CompilerParams(use_tc_tiling_on_sc=False),
    )
    def gather_item(idx_hbm, table_hbm, out_hbm, idx_v, rows0, rows1, gsem, ssem):
        wid = lax.axis_index("s") * _NC + lax.axis_index("c")
        base = wid * _B_PER_W
        pltpu.sync_copy(idx_hbm.at[pl.ds(base, _B_PER_W)], idx_v)
        bufs = (rows0, rows1)
        gathers = [None] * _N_CHUNKS
        scatters = [None] * _N_CHUNKS
        gathers[0] = pltpu.async_copy(
            table_hbm.at[idx_v.at[pl.ds(0, _CHUNK)]], bufs[0], gsem
        )
        for c in range(_N_CHUNKS):
            buf = bufs[c % 2]
            gathers[c].wait()
            scatters[c] = pltpu.async_copy(
                buf.at[:, pl.ds(0, D)],
                out_hbm.at[pl.ds(base + c * _CHUNK, _CHUNK)],
                ssem,
            )
            nxt = c + 1
            if nxt < _N_CHUNKS:
                if nxt >= 2:
                    scatters[nxt - 2].wait()
                gathers[nxt] = pltpu.async_copy(
                    table_hbm.at[idx_v.at[pl.ds(nxt * _CHUNK, _CHUNK)]],
                    bufs[nxt % 2],
                    gsem,
                )
        scatters[_N_CHUNKS - 2].wait()
        scatters[_N_CHUNKS - 1].wait()

    return gather_item


_L_BLK = 5


def _small_lookup_tc_body(ha_i, ma_i, sa_i, hb_i, mb_i, sb_i,
                          ht_a, mt_a, st_a, ht_b, mt_b, st_b,
                          o1, o2, o3, o4, o5, o6):
    pairs = [
        (ha_i, ht_a, o1), (ma_i, mt_a, o2), (sa_i, st_a, o3),
        (hb_i, ht_b, o4), (mb_i, mt_b, o5), (sb_i, st_b, o6),
    ]
    l0 = pl.program_id(0) * _L_BLK
    for idx_ref, tab_ref, out_ref in pairs:
        t_rows = tab_ref.shape[0]
        tab = tab_ref[...]
        iota = lax.broadcasted_iota(jnp.int32, (t_rows, B), 0)
        for j in range(_L_BLK):
            row = idx_ref[l0 + j, :]
            onehot = jnp.where(iota == row[None, :], 1.0, 0.0).astype(jnp.float32)
            out_ref[j, :, :] = lax.dot_general(
                tab, onehot,
                ((( 0,), (0,)), ((), ())),
                preferred_element_type=jnp.float32,
                precision=lax.Precision.HIGHEST,
            )


def _small_lookup_tc(idxTs, tables):
    idx_spec = pl.BlockSpec((L, B), lambda i: (0, 0))
    tab_specs = [
        pl.BlockSpec((t.shape[0], D), lambda i: (0, 0)) for t in tables
    ]
    out_spec = pl.BlockSpec((_L_BLK, D, B), lambda i: (i, 0, 0))
    out_struct = jax.ShapeDtypeStruct((L, D, B), jnp.float32)
    return pl.pallas_call(
        _small_lookup_tc_body,
        grid=(L // _L_BLK,),
        in_specs=[idx_spec] * 6 + tab_specs,
        out_specs=[out_spec] * 6,
        out_shape=[out_struct] * 6,
    )(*idxTs, *tables)


def kernel(session, h_a_o, m_a_o, s_a_o, h_b_o, m_b_o, s_b_o,
           item_table, hour_table_a, minute_table_a, second_table_a,
           hour_table_b, minute_table_b, second_table_b):
    table_pad = jnp.pad(item_table, ((0, 7), (0, _DP - D)))
    item_out = _item_gather_sc()(session.reshape(N_IDX), table_pad)

    idxTs = [a.T for a in (h_a_o, m_a_o, s_a_o, h_b_o, m_b_o, s_b_o)]
    tables = [hour_table_a, minute_table_a, second_table_a,
              hour_table_b, minute_table_b, second_table_b]
    small = _small_lookup_tc(idxTs, tables)

    outs = [item_out.reshape(B, L, D)]
    outs += [jnp.transpose(y, (2, 0, 1)) for y in small]
    return tuple(outs)
